# baseline (speedup 1.0000x reference)
import functools

import jax
import jax.numpy as jnp
from jax import lax
from jax.experimental import pallas as pl
from jax.experimental.pallas import tpu as pltpu
from jax.experimental.pallas import tpu_sc as plsc

_N = 10000
_H = 4
_DH = 32
_F = _H * _DH
_ROW = 144
_DROW = 16
_NEG = 0.2
_BN = 1000
_NC = 2
_NS = 16
_K = 80
_NP = 10000
_RPT = _NP // _NS
_E = 320000
_CLIP = 60.0


def _lrelu(v):
    return jnp.where(v >= 0, v, _NEG * v)


def _project_body(h_ref, p_ref, gw_ref, gb_ref, pw_ref, lw_ref, aa_ref, ab_ref,
                  xsrc_ref, dpk_ref):
    hb = h_ref[...]
    pb = p_ref[...]
    g = jnp.dot(hb, gw_ref[...], preferred_element_type=jnp.float32) + gb_ref[...]
    pf = jnp.dot(pb, pw_ref[...], preferred_element_type=jnp.float32)
    xf = jnp.dot(hb, lw_ref[...], preferred_element_type=jnp.float32)
    ai = jnp.sum((pf * aa_ref[...]).reshape(_BN, _H, _DH), axis=-1) * g
    aj = jnp.sum((pf * ab_ref[...]).reshape(_BN, _H, _DH), axis=-1) * g
    ss = _lrelu(ai + aj)
    z = jnp.zeros((_BN, _ROW - _F - 2 * _H), jnp.float32)
    xsrc_ref[...] = jnp.concatenate([xf, ai, ss, z], axis=1)
    dpk_ref[...] = jnp.concatenate(
        [aj, jnp.zeros((_BN, _DROW - _H), jnp.float32)], axis=1)


def _project(h, persona, gw, gb, pw, lw, aa, ab):
    nb = _N // _BN
    return pl.pallas_call(
        _project_body,
        grid=(nb,),
        in_specs=[
            pl.BlockSpec((_BN, _F), lambda i: (i, 0)),
            pl.BlockSpec((_BN, _F), lambda i: (i, 0)),
            pl.BlockSpec((_F, _H), lambda i: (0, 0)),
            pl.BlockSpec((1, _H), lambda i: (0, 0)),
            pl.BlockSpec((_F, _F), lambda i: (0, 0)),
            pl.BlockSpec((_F, _F), lambda i: (0, 0)),
            pl.BlockSpec((1, _F), lambda i: (0, 0)),
            pl.BlockSpec((1, _F), lambda i: (0, 0)),
        ],
        out_specs=[
            pl.BlockSpec((_BN, _ROW), lambda i: (i, 0)),
            pl.BlockSpec((_BN, _DROW), lambda i: (i, 0)),
        ],
        out_shape=[
            jax.ShapeDtypeStruct((_N, _ROW), jnp.float32),
            jax.ShapeDtypeStruct((_N, _DROW), jnp.float32),
        ],
    )(h, persona, gw, gb, pw, lw, aa, ab)


def _edge_kernel_body(xsrc_hbm, dpk_hbm, src_hbm, dst2_hbm, out_hbm,
                      sidx0, sidx1, didx_all, rows0, rows1, dpks0, dpks1,
                      wbuf, acc_sh, is0, is1, gs0, gs1, ss0, ss1):
    sidxb = (sidx0, sidx1)
    rowsb = (rows0, rows1)
    dpksb = (dpks0, dpks1)
    isem = (is0, is1)
    gsem = (gs0, gs1)
    ssem = (ss0, ss1)
    nch = dst2_hbm.shape[0] // (_NC * _NS)
    ept = nch * _K
    cid = lax.axis_index("c")
    sid = lax.axis_index("s")
    wid = cid * _NS + sid
    lane = jnp.arange(16, dtype=jnp.int32)
    zero16 = jnp.zeros((16,), jnp.float32)

    ibase = pl.multiple_of(wid * nch, nch)
    pltpu.sync_copy(dst2_hbm.at[pl.ds(ibase, nch)], didx_all)

    for j in range(_K * 8 // 16):
        wbuf[pl.ds(j * 16, 16)] = zero16

    def _zb_row(i, _):
        for j in range(_ROW // 16):
            rows0[i, pl.ds(j * 16, 16)] = zero16
        return 0
    lax.fori_loop(0, _K, _zb_row, 0)
    nfull = _RPT // _K
    for r in range(nfull):
        pltpu.sync_copy(
            rows0, acc_sh.at[pl.ds(pl.multiple_of(sid * _RPT + r * _K, 1), _K)])
    rem = _RPT - nfull * _K
    if rem:
        pltpu.sync_copy(
            rows0.at[pl.ds(0, rem)],
            acc_sh.at[pl.ds(pl.multiple_of(sid * _RPT + nfull * _K, 1), rem)])
    plsc.subcore_barrier()

    pat8 = jnp.where(lane < _H, lane, 4).astype(jnp.int32)
    hvec = [jnp.full((16,), h, jnp.int32) for h in range(_H)]
    base_e = wid * ept

    def istart(c, b):
        off = pl.multiple_of(base_e + c * _K, 8)
        pltpu.async_copy(src_hbm.at[pl.ds(off, _K)], sidxb[b], isem[b])

    def iwait(c, b):
        off = pl.multiple_of(base_e + c * _K, 8)
        pltpu.make_async_copy(src_hbm.at[pl.ds(off, _K)], sidxb[b], isem[b]).wait()

    def gather_start(c, b):
        pltpu.async_copy(xsrc_hbm.at[sidxb[b]], rowsb[b], gsem[b])
        pltpu.async_copy(dpk_hbm.at[didx_all.at[c]], dpksb[b], gsem[b])

    def gather_wait(c, b):
        pltpu.make_async_copy(xsrc_hbm.at[sidxb[b]], rowsb[b], gsem[b]).wait()
        pltpu.make_async_copy(dpk_hbm.at[didx_all.at[c]], dpksb[b], gsem[b]).wait()

    def scatter_start(c, b):
        pltpu.async_copy(rowsb[b], acc_sh.at[didx_all.at[c]], ssem[b], add=True)

    def scatter_wait(c, b):
        pltpu.make_async_copy(rowsb[b], acc_sh.at[didx_all.at[c]], ssem[b]).wait()

    def compute(c, b):
        rows = rowsb[b]
        dpks = dpksb[b]
        sidx = sidxb[b]
        cv = jnp.full((16,), 0, jnp.int32) + c

        def _score(g, _):
            e16 = g * 16 + lane
            sv = plsc.load_gather(sidx, [e16])
            dv = plsc.load_gather(didx_all, [cv, e16])
            m = sv != dv
            for h in range(_H):
                ai = plsc.load_gather(rows, [e16, hvec[h] + _F])
                aj = plsc.load_gather(dpks, [e16, hvec[h]])
                s = _lrelu(ai + aj)
                w = jnp.exp(jnp.clip(s, -_CLIP, _CLIP))
                w = jnp.where(m, w, 0.0)
                plsc.store_scatter(wbuf, [e16 * 8 + h], w)
            return 0
        lax.fori_loop(0, _K // 16, _score, 0)

        def _mul(i, _):
            for u in range(2):
                e = i * 2 + u
                for h in range(_H):
                    wp = plsc.load_gather(wbuf, [e * 8 + hvec[h]])
                    for j in (2 * h, 2 * h + 1):
                        rows[e, pl.ds(j * 16, 16)] = (
                            wp * rows[e, pl.ds(j * 16, 16)])
                rows[e, pl.ds(8 * 16, 16)] = plsc.load_gather(wbuf, [e * 8 + pat8])
            return 0
        lax.fori_loop(0, _K // 2, _mul, 0)

    def step(c, b, guard, last):
        gather_wait(c, b)
        if not last:
            iwait(c + 1, 1 - b)
            gather_start(c + 1, 1 - b)
        compute(c, b)
        if not last:
            @pl.when(c + 2 < nch)
            def _():
                istart(c + 2, b)
        pltpu.sync_copy(rowsb[b], acc_sh.at[didx_all.at[c]], add=True)

    istart(0, 0)
    iwait(0, 0)
    gather_start(0, 0)
    istart(1, 1)

    def _pipe(t, _):
        step(2 * t, 0, True, False)
        step(2 * t + 1, 1, False, False)
        return 0
    lax.fori_loop(0, (nch - 1) // 2, _pipe, 0)
    step(nch - 1, (nch - 1) % 2, False, True)

    plsc.subcore_barrier()
    obase = pl.multiple_of(sid * _RPT, 1)
    pltpu.sync_copy(acc_sh.at[pl.ds(obase, _RPT)],
                    out_hbm.at[cid, pl.ds(obase, _RPT)])


def _edge_pass(xsrc, dpk, src, dst):
    mesh = plsc.VectorSubcoreMesh(core_axis_name="c", subcore_axis_name="s",
                                  num_cores=_NC, num_subcores=_NS)
    fn = functools.partial(
        pl.kernel,
        out_type=jax.ShapeDtypeStruct((_NC, _NP, _ROW), jnp.float32),
        mesh=mesh,
        compiler_params=pltpu.CompilerParams(use_tc_tiling_on_sc=False,
                                             needs_layout_passes=False),
        scratch_types=[
            pltpu.VMEM((_K,), jnp.int32),
            pltpu.VMEM((_K,), jnp.int32),
            pltpu.VMEM((_E // _K // (_NC * _NS), _K), jnp.int32),
            pltpu.VMEM((_K, _ROW), jnp.float32),
            pltpu.VMEM((_K, _ROW), jnp.float32),
            pltpu.VMEM((_K, _DROW), jnp.float32),
            pltpu.VMEM((_K, _DROW), jnp.float32),
            pltpu.VMEM((_K * 8,), jnp.float32),
            pltpu.VMEM_SHARED((_NP, _ROW), jnp.float32),
            pltpu.SemaphoreType.DMA,
            pltpu.SemaphoreType.DMA,
            pltpu.SemaphoreType.DMA,
            pltpu.SemaphoreType.DMA,
            pltpu.SemaphoreType.DMA,
            pltpu.SemaphoreType.DMA,
        ],
    )(_edge_kernel_body)
    return fn(xsrc, dpk, src, dst.reshape(_E // _K, _K))


def _combine_body(h_ref, xsrc_ref, a0_ref, a1_ref, b_ref, out_ref):
    xs = xsrc_ref[...]
    a0 = a0_ref[...]
    a1 = a1_ref[...]
    ess = jnp.exp(jnp.clip(xs[:, _F + _H:_F + 2 * _H], -_CLIP, _CLIP))
    essb = jnp.broadcast_to(ess[:, :, None], (_BN, _H, _DH)).reshape(_BN, _F)
    num = essb * xs[:, :_F] + a0[:, :_F] + a1[:, :_F]
    den = ess + a0[:, _F:_F + _H] + a1[:, _F:_F + _H]
    denb = jnp.broadcast_to(den[:, :, None], (_BN, _H, _DH)).reshape(_BN, _F)
    o = num / denb + b_ref[...]
    o = jnp.where(o > 0, o, jnp.exp(jnp.minimum(o, 0.0)) - 1.0)
    out_ref[...] = h_ref[...] + o


def _combine(h, xsrc, acc0, acc1, b):
    nb = _N // _BN
    return pl.pallas_call(
        _combine_body,
        grid=(nb,),
        in_specs=[
            pl.BlockSpec((_BN, _F), lambda i: (i, 0)),
            pl.BlockSpec((_BN, _ROW), lambda i: (i, 0)),
            pl.BlockSpec((_BN, _ROW), lambda i: (i, 0)),
            pl.BlockSpec((_BN, _ROW), lambda i: (i, 0)),
            pl.BlockSpec((1, _F), lambda i: (0, 0)),
        ],
        out_specs=pl.BlockSpec((_BN, _F), lambda i: (i, 0)),
        out_shape=jax.ShapeDtypeStruct((_N, _F), jnp.float32),
    )(h, xsrc, acc0, acc1, b)


def kernel(x, persona, edge_index, gate_W, gate_b, persona_W, lin_W, att_W, bias):
    src = edge_index[0]
    dst = edge_index[1]
    h = x
    L = gate_W.shape[0]
    for l in range(L):
        gw = gate_W[l, :, :, 0].T
        gb = gate_b[l, :, 0][None, :]
        pw = persona_W[l].transpose(1, 0, 2).reshape(_F, _F)
        lw = lin_W[l].transpose(1, 0, 2).reshape(_F, _F)
        aa = att_W[l, :, :_DH, 0].reshape(1, _F)
        ab = att_W[l, :, _DH:, 0].reshape(1, _F)
        bl = bias[l][None, :]
        xsrc, dpk = _project(h, persona, gw, gb, pw, lw, aa, ab)
        acc = _edge_pass(xsrc, dpk, src, dst)
        h = _combine(h, xsrc, acc[0, :_N], acc[1, :_N], bl)
    return h


# speedup vs baseline: 1.5421x; 1.5421x over previous
import functools

import jax
import jax.numpy as jnp
from jax import lax
from jax.experimental import pallas as pl
from jax.experimental.pallas import tpu as pltpu
from jax.experimental.pallas import tpu_sc as plsc

_N = 10000
_H = 4
_DH = 32
_F = _H * _DH
_ROW = 144
_DROW = 16
_NEG = 0.2
_BN = 1000
_NC = 2
_NS = 16
_K = 80
_NP = 10000
_RPT = _NP // _NS
_E = 320000
_CLIP = 60.0


def _lrelu(v):
    return jnp.where(v >= 0, v, _NEG * v)


def _project_body(h_ref, p_ref, gw_ref, gb_ref, pw_ref, lw_ref, aa_ref, ab_ref,
                  xsrc_ref, dpk_ref):
    hb = h_ref[...]
    pb = p_ref[...]
    g = jnp.dot(hb, gw_ref[...], preferred_element_type=jnp.float32) + gb_ref[...]
    pf = jnp.dot(pb, pw_ref[...], preferred_element_type=jnp.float32)
    xf = jnp.dot(hb, lw_ref[...], preferred_element_type=jnp.float32)
    ai = jnp.sum((pf * aa_ref[...]).reshape(_BN, _H, _DH), axis=-1) * g
    aj = jnp.sum((pf * ab_ref[...]).reshape(_BN, _H, _DH), axis=-1) * g
    ss = _lrelu(ai + aj)
    z = jnp.zeros((_BN, _ROW - _F - 2 * _H), jnp.float32)
    xsrc_ref[...] = jnp.concatenate([xf, ai, ss, z], axis=1)
    dpk_ref[...] = jnp.concatenate(
        [aj, jnp.zeros((_BN, _DROW - _H), jnp.float32)], axis=1)


def _project(h, persona, gw, gb, pw, lw, aa, ab):
    nb = _N // _BN
    return pl.pallas_call(
        _project_body,
        grid=(nb,),
        in_specs=[
            pl.BlockSpec((_BN, _F), lambda i: (i, 0)),
            pl.BlockSpec((_BN, _F), lambda i: (i, 0)),
            pl.BlockSpec((_F, _H), lambda i: (0, 0)),
            pl.BlockSpec((1, _H), lambda i: (0, 0)),
            pl.BlockSpec((_F, _F), lambda i: (0, 0)),
            pl.BlockSpec((_F, _F), lambda i: (0, 0)),
            pl.BlockSpec((1, _F), lambda i: (0, 0)),
            pl.BlockSpec((1, _F), lambda i: (0, 0)),
        ],
        out_specs=[
            pl.BlockSpec((_BN, _ROW), lambda i: (i, 0)),
            pl.BlockSpec((_BN, _DROW), lambda i: (i, 0)),
        ],
        out_shape=[
            jax.ShapeDtypeStruct((_N, _ROW), jnp.float32),
            jax.ShapeDtypeStruct((_N, _DROW), jnp.float32),
        ],
    )(h, persona, gw, gb, pw, lw, aa, ab)


def _edge_kernel_body(xsrc_hbm, dpk_hbm, src_hbm, dst2_hbm, out_hbm,
                      sidx0, sidx1, didx_all, rows0, rows1, dpks0, dpks1,
                      wbuf, acc_sh, is0, is1, gs0, gs1, ss0, ss1):
    sidxb = (sidx0, sidx1)
    rowsb = (rows0, rows1)
    dpksb = (dpks0, dpks1)
    isem = (is0, is1)
    gsem = (gs0, gs1)
    ssem = (ss0, ss1)
    nch = dst2_hbm.shape[0] // (_NC * _NS)
    ept = nch * _K
    cid = lax.axis_index("c")
    sid = lax.axis_index("s")
    wid = cid * _NS + sid
    lane = jnp.arange(16, dtype=jnp.int32)
    zero16 = jnp.zeros((16,), jnp.float32)

    ibase = pl.multiple_of(wid * nch, nch)
    pltpu.sync_copy(dst2_hbm.at[pl.ds(ibase, nch)], didx_all)

    for j in range(_K * 8 // 16):
        wbuf[pl.ds(j * 16, 16)] = zero16

    def _zb_row(i, _):
        for j in range(_ROW // 16):
            rows0[i, pl.ds(j * 16, 16)] = zero16
        return 0
    lax.fori_loop(0, _K, _zb_row, 0)
    nfull = _RPT // _K
    for r in range(nfull):
        pltpu.sync_copy(
            rows0, acc_sh.at[pl.ds(pl.multiple_of(sid * _RPT + r * _K, 1), _K)])
    rem = _RPT - nfull * _K
    if rem:
        pltpu.sync_copy(
            rows0.at[pl.ds(0, rem)],
            acc_sh.at[pl.ds(pl.multiple_of(sid * _RPT + nfull * _K, 1), rem)])
    plsc.subcore_barrier()

    pat8 = jnp.where(lane < _H, lane, 4).astype(jnp.int32)
    hvec = [jnp.full((16,), h, jnp.int32) for h in range(_H)]
    base_e = wid * ept

    def istart(c, b):
        off = pl.multiple_of(base_e + c * _K, 8)
        pltpu.async_copy(src_hbm.at[pl.ds(off, _K)], sidxb[b], isem[b])

    def iwait(c, b):
        off = pl.multiple_of(base_e + c * _K, 8)
        pltpu.make_async_copy(src_hbm.at[pl.ds(off, _K)], sidxb[b], isem[b]).wait()

    def gather_start(c, b):
        pltpu.async_copy(xsrc_hbm.at[sidxb[b]], rowsb[b], gsem[b])
        pltpu.async_copy(dpk_hbm.at[didx_all.at[c]], dpksb[b], gsem[b])

    def gather_wait(c, b):
        pltpu.make_async_copy(xsrc_hbm.at[sidxb[b]], rowsb[b], gsem[b]).wait()
        pltpu.make_async_copy(dpk_hbm.at[didx_all.at[c]], dpksb[b], gsem[b]).wait()

    def scatter_start(c, b):
        pltpu.async_copy(rowsb[b], acc_sh.at[didx_all.at[c]], ssem[b], add=True)

    def scatter_wait(c, b):
        pltpu.make_async_copy(rowsb[b], acc_sh.at[didx_all.at[c]], ssem[b]).wait()

    def compute(c, b):
        rows = rowsb[b]
        dpks = dpksb[b]
        sidx = sidxb[b]
        cv = jnp.full((16,), 0, jnp.int32) + c

        @plsc.parallel_loop(0, _K // 16)
        def _score(g):
            e16 = g * 16 + lane
            sv = plsc.load_gather(sidx, [e16])
            dv = plsc.load_gather(didx_all, [cv, e16])
            m = sv != dv
            for h in range(_H):
                ai = plsc.load_gather(rows, [e16, hvec[h] + _F])
                aj = plsc.load_gather(dpks, [e16, hvec[h]])
                s = _lrelu(ai + aj)
                w = jnp.exp(jnp.clip(s, -_CLIP, _CLIP))
                w = jnp.where(m, w, 0.0)
                plsc.store_scatter(wbuf, [e16 * 8 + h], w)

        @plsc.parallel_loop(0, _K, unroll=2)
        def _mul(e):
            for h in range(_H):
                wp = plsc.load_gather(wbuf, [e * 8 + hvec[h]])
                for j in (2 * h, 2 * h + 1):
                    rows[e, pl.ds(j * 16, 16)] = wp * rows[e, pl.ds(j * 16, 16)]
            rows[e, pl.ds(8 * 16, 16)] = plsc.load_gather(wbuf, [e * 8 + pat8])

    def step(c, b, guard, last):
        gather_wait(c, b)
        if not last:
            iwait(c + 1, 1 - b)
            gather_start(c + 1, 1 - b)
        compute(c, b)
        if not last:
            @pl.when(c + 2 < nch)
            def _():
                istart(c + 2, b)
        pltpu.sync_copy(rowsb[b], acc_sh.at[didx_all.at[c]], add=True)

    istart(0, 0)
    iwait(0, 0)
    gather_start(0, 0)
    istart(1, 1)

    def _pipe(t, _):
        step(2 * t, 0, True, False)
        step(2 * t + 1, 1, False, False)
        return 0
    lax.fori_loop(0, (nch - 1) // 2, _pipe, 0)
    step(nch - 1, (nch - 1) % 2, False, True)

    plsc.subcore_barrier()
    obase = pl.multiple_of(sid * _RPT, 1)
    pltpu.sync_copy(acc_sh.at[pl.ds(obase, _RPT)],
                    out_hbm.at[cid, pl.ds(obase, _RPT)])


def _edge_pass(xsrc, dpk, src, dst):
    mesh = plsc.VectorSubcoreMesh(core_axis_name="c", subcore_axis_name="s",
                                  num_cores=_NC, num_subcores=_NS)
    fn = functools.partial(
        pl.kernel,
        out_type=jax.ShapeDtypeStruct((_NC, _NP, _ROW), jnp.float32),
        mesh=mesh,
        compiler_params=pltpu.CompilerParams(use_tc_tiling_on_sc=False,
                                             needs_layout_passes=False),
        scratch_types=[
            pltpu.VMEM((_K,), jnp.int32),
            pltpu.VMEM((_K,), jnp.int32),
            pltpu.VMEM((_E // _K // (_NC * _NS), _K), jnp.int32),
            pltpu.VMEM((_K, _ROW), jnp.float32),
            pltpu.VMEM((_K, _ROW), jnp.float32),
            pltpu.VMEM((_K, _DROW), jnp.float32),
            pltpu.VMEM((_K, _DROW), jnp.float32),
            pltpu.VMEM((_K * 8,), jnp.float32),
            pltpu.VMEM_SHARED((_NP, _ROW), jnp.float32),
            pltpu.SemaphoreType.DMA,
            pltpu.SemaphoreType.DMA,
            pltpu.SemaphoreType.DMA,
            pltpu.SemaphoreType.DMA,
            pltpu.SemaphoreType.DMA,
            pltpu.SemaphoreType.DMA,
        ],
    )(_edge_kernel_body)
    return fn(xsrc, dpk, src, dst.reshape(_E // _K, _K))


def _combine_body(h_ref, xsrc_ref, a0_ref, a1_ref, b_ref, out_ref):
    xs = xsrc_ref[...]
    a0 = a0_ref[...]
    a1 = a1_ref[...]
    ess = jnp.exp(jnp.clip(xs[:, _F + _H:_F + 2 * _H], -_CLIP, _CLIP))
    essb = jnp.broadcast_to(ess[:, :, None], (_BN, _H, _DH)).reshape(_BN, _F)
    num = essb * xs[:, :_F] + a0[:, :_F] + a1[:, :_F]
    den = ess + a0[:, _F:_F + _H] + a1[:, _F:_F + _H]
    denb = jnp.broadcast_to(den[:, :, None], (_BN, _H, _DH)).reshape(_BN, _F)
    o = num / denb + b_ref[...]
    o = jnp.where(o > 0, o, jnp.exp(jnp.minimum(o, 0.0)) - 1.0)
    out_ref[...] = h_ref[...] + o


def _combine(h, xsrc, acc0, acc1, b):
    nb = _N // _BN
    return pl.pallas_call(
        _combine_body,
        grid=(nb,),
        in_specs=[
            pl.BlockSpec((_BN, _F), lambda i: (i, 0)),
            pl.BlockSpec((_BN, _ROW), lambda i: (i, 0)),
            pl.BlockSpec((_BN, _ROW), lambda i: (i, 0)),
            pl.BlockSpec((_BN, _ROW), lambda i: (i, 0)),
            pl.BlockSpec((1, _F), lambda i: (0, 0)),
        ],
        out_specs=pl.BlockSpec((_BN, _F), lambda i: (i, 0)),
        out_shape=jax.ShapeDtypeStruct((_N, _F), jnp.float32),
    )(h, xsrc, acc0, acc1, b)


def kernel(x, persona, edge_index, gate_W, gate_b, persona_W, lin_W, att_W, bias):
    src = edge_index[0]
    dst = edge_index[1]
    h = x
    L = gate_W.shape[0]
    for l in range(L):
        gw = gate_W[l, :, :, 0].T
        gb = gate_b[l, :, 0][None, :]
        pw = persona_W[l].transpose(1, 0, 2).reshape(_F, _F)
        lw = lin_W[l].transpose(1, 0, 2).reshape(_F, _F)
        aa = att_W[l, :, :_DH, 0].reshape(1, _F)
        ab = att_W[l, :, _DH:, 0].reshape(1, _F)
        bl = bias[l][None, :]
        xsrc, dpk = _project(h, persona, gw, gb, pw, lw, aa, ab)
        acc = _edge_pass(xsrc, dpk, src, dst)
        h = _combine(h, xsrc, acc[0, :_N], acc[1, :_N], bl)
    return h


# speedup vs baseline: 1.6628x; 1.0783x over previous
import functools

import jax
import jax.numpy as jnp
from jax import lax
from jax.experimental import pallas as pl
from jax.experimental.pallas import tpu as pltpu
from jax.experimental.pallas import tpu_sc as plsc

_N = 10000
_H = 4
_DH = 32
_F = _H * _DH
_ROW = 144
_DROW = 16
_NEG = 0.2
_BN = 1000
_NC = 2
_NS = 16
_K = 80
_NP = 10000
_RPT = _NP // _NS
_E = 320000
_CLIP = 60.0


def _lrelu(v):
    return jnp.where(v >= 0, v, _NEG * v)


def _project_body(h_ref, p_ref, gw_ref, gb_ref, pw_ref, lw_ref, aa_ref, ab_ref,
                  xsrc_ref, dpk_ref):
    hb = h_ref[...]
    pb = p_ref[...]
    g = jnp.dot(hb, gw_ref[...], preferred_element_type=jnp.float32) + gb_ref[...]
    pf = jnp.dot(pb, pw_ref[...], preferred_element_type=jnp.float32)
    xf = jnp.dot(hb, lw_ref[...], preferred_element_type=jnp.float32)
    ai = jnp.sum((pf * aa_ref[...]).reshape(_BN, _H, _DH), axis=-1) * g
    aj = jnp.sum((pf * ab_ref[...]).reshape(_BN, _H, _DH), axis=-1) * g
    ss = _lrelu(ai + aj)
    z = jnp.zeros((_BN, _ROW - _F - 2 * _H), jnp.float32)
    xsrc_ref[...] = jnp.concatenate([xf, ai, ss, z], axis=1)
    dpk_ref[...] = jnp.concatenate(
        [aj, jnp.zeros((_BN, _DROW - _H), jnp.float32)], axis=1)


def _project(h, persona, gw, gb, pw, lw, aa, ab):
    nb = _N // _BN
    return pl.pallas_call(
        _project_body,
        grid=(nb,),
        in_specs=[
            pl.BlockSpec((_BN, _F), lambda i: (i, 0)),
            pl.BlockSpec((_BN, _F), lambda i: (i, 0)),
            pl.BlockSpec((_F, _H), lambda i: (0, 0)),
            pl.BlockSpec((1, _H), lambda i: (0, 0)),
            pl.BlockSpec((_F, _F), lambda i: (0, 0)),
            pl.BlockSpec((_F, _F), lambda i: (0, 0)),
            pl.BlockSpec((1, _F), lambda i: (0, 0)),
            pl.BlockSpec((1, _F), lambda i: (0, 0)),
        ],
        out_specs=[
            pl.BlockSpec((_BN, _ROW), lambda i: (i, 0)),
            pl.BlockSpec((_BN, _DROW), lambda i: (i, 0)),
        ],
        out_shape=[
            jax.ShapeDtypeStruct((_N, _ROW), jnp.float32),
            jax.ShapeDtypeStruct((_N, _DROW), jnp.float32),
        ],
    )(h, persona, gw, gb, pw, lw, aa, ab)


def _edge_kernel_body(xsrc_hbm, dpk_hbm, src_hbm, dst2_hbm, out_hbm,
                      si0, si1, si2, si3, di0, di1, di2, di3,
                      rows0, rows1, rows2, dp0, dp1, dp2,
                      wbuf, acc_sh,
                      is0, is1, is2, is3, gs0, gs1, gs2, ss0, ss1, ss2):
    sidx4 = (si0, si1, si2, si3)
    didx4 = (di0, di1, di2, di3)
    rows3 = (rows0, rows1, rows2)
    dpks3 = (dp0, dp1, dp2)
    isem = (is0, is1, is2, is3)
    gsem = (gs0, gs1, gs2)
    ssem = (ss0, ss1, ss2)
    nch = dst2_hbm.shape[0] // (_NC * _NS)
    ept = nch * _K
    cid = lax.axis_index("c")
    sid = lax.axis_index("s")
    wid = cid * _NS + sid
    lane = jnp.arange(16, dtype=jnp.int32)
    zero16 = jnp.zeros((16,), jnp.float32)

    for j in range(_K * 8 // 16):
        wbuf[pl.ds(j * 16, 16)] = zero16

    def _zb_row(i, _):
        for j in range(_ROW // 16):
            rows0[i, pl.ds(j * 16, 16)] = zero16
        return 0
    lax.fori_loop(0, _K, _zb_row, 0)
    nfull = _RPT // _K
    for r in range(nfull):
        pltpu.sync_copy(
            rows0, acc_sh.at[pl.ds(pl.multiple_of(sid * _RPT + r * _K, 1), _K)])
    rem = _RPT - nfull * _K
    if rem:
        pltpu.sync_copy(
            rows0.at[pl.ds(0, rem)],
            acc_sh.at[pl.ds(pl.multiple_of(sid * _RPT + nfull * _K, 1), rem)])
    plsc.subcore_barrier()

    pat8 = jnp.where(lane < _H, lane, 4).astype(jnp.int32)
    hvec = [jnp.full((16,), h, jnp.int32) for h in range(_H)]
    zvec = jnp.zeros((16,), jnp.int32)
    base_e = wid * ept

    def istart(c, k):
        off = pl.multiple_of(base_e + c * _K, 8)
        pltpu.async_copy(src_hbm.at[pl.ds(off, _K)], sidx4[k], isem[k])
        crow = pl.multiple_of(wid * nch + c, 1)
        pltpu.async_copy(dst2_hbm.at[pl.ds(crow, 1)], didx4[k], isem[k])

    def iwait(c, k):
        off = pl.multiple_of(base_e + c * _K, 8)
        pltpu.make_async_copy(src_hbm.at[pl.ds(off, _K)], sidx4[k], isem[k]).wait()
        crow = pl.multiple_of(wid * nch + c, 1)
        pltpu.make_async_copy(dst2_hbm.at[pl.ds(crow, 1)], didx4[k], isem[k]).wait()

    def gather_start(b, k):
        pltpu.async_copy(xsrc_hbm.at[sidx4[k]], rows3[b], gsem[b])
        pltpu.async_copy(dpk_hbm.at[didx4[k].at[0]], dpks3[b], gsem[b])

    def gather_wait(b, k):
        pltpu.make_async_copy(xsrc_hbm.at[sidx4[k]], rows3[b], gsem[b]).wait()
        pltpu.make_async_copy(dpk_hbm.at[didx4[k].at[0]], dpks3[b], gsem[b]).wait()

    def scatter_start(b, k):
        pltpu.async_copy(rows3[b], acc_sh.at[didx4[k].at[0]], ssem[b], add=True)

    def scatter_wait(b, k):
        pltpu.make_async_copy(rows3[b], acc_sh.at[didx4[k].at[0]], ssem[b]).wait()

    def compute(b, k):
        rows = rows3[b]
        dpks = dpks3[b]
        sidx = sidx4[k]
        didx = didx4[k]

        @plsc.parallel_loop(0, _K // 16)
        def _score(g):
            e16 = g * 16 + lane
            sv = plsc.load_gather(sidx, [e16])
            dv = plsc.load_gather(didx, [zvec, e16])
            m = sv != dv
            for h in range(_H):
                ai = plsc.load_gather(rows, [e16, hvec[h] + _F])
                aj = plsc.load_gather(dpks, [e16, hvec[h]])
                sc = _lrelu(ai + aj)
                w = jnp.exp(jnp.clip(sc, -_CLIP, _CLIP))
                w = jnp.where(m, w, 0.0)
                plsc.store_scatter(wbuf, [e16 * 8 + h], w)

        @plsc.parallel_loop(0, _K, unroll=2)
        def _mul(e):
            for h in range(_H):
                wp = plsc.load_gather(wbuf, [e * 8 + hvec[h]])
                for j in (2 * h, 2 * h + 1):
                    rows[e, pl.ds(j * 16, 16)] = wp * rows[e, pl.ds(j * 16, 16)]
            rows[e, pl.ds(8 * 16, 16)] = plsc.load_gather(wbuf, [e * 8 + pat8])

    def step(c, j, guard0, tail):
        b3, b4 = j % 3, j % 4
        pv3, pv4 = (j + 2) % 3, (j + 3) % 4
        bn3, bn4 = (j + 2) % 3, (j + 2) % 4
        nx4 = (j + 3) % 4
        gather_wait(b3, b4)
        compute(b3, b4)
        scatter_start(b3, b4)
        if guard0:
            @pl.when(c > 0)
            def _():
                scatter_wait(pv3, pv4)
        else:
            scatter_wait(pv3, pv4)
        if not tail or c + 2 < nch:
            iwait(c + 2, bn4)
            gather_start(bn3, bn4)
        if not tail or c + 3 < nch:
            istart(c + 3, nx4)

    nmain = (nch - 5) // 12
    istart(0, 0)
    istart(1, 1)
    istart(2, 2)
    iwait(0, 0)
    gather_start(0, 0)
    iwait(1, 1)
    gather_start(1, 1)

    def _pipe(t, _):
        c0 = t * 12
        for j in range(12):
            step(c0 + j, j, j == 0, False)
        return 0
    lax.fori_loop(0, nmain, _pipe, 0)
    for c in range(nmain * 12, nch):
        step(c, c % 12, False, True)
    scatter_wait((nch - 1) % 3, (nch - 1) % 4)

    plsc.subcore_barrier()
    obase = pl.multiple_of(sid * _RPT, 1)
    pltpu.sync_copy(acc_sh.at[pl.ds(obase, _RPT)],
                    out_hbm.at[cid, pl.ds(obase, _RPT)])


def _edge_pass(xsrc, dpk, src, dst):
    mesh = plsc.VectorSubcoreMesh(core_axis_name="c", subcore_axis_name="s",
                                  num_cores=_NC, num_subcores=_NS)
    fn = functools.partial(
        pl.kernel,
        out_type=jax.ShapeDtypeStruct((_NC, _NP, _ROW), jnp.float32),
        mesh=mesh,
        compiler_params=pltpu.CompilerParams(use_tc_tiling_on_sc=False,
                                             needs_layout_passes=False),
        scratch_types=(
            [pltpu.VMEM((_K,), jnp.int32)] * 4
            + [pltpu.VMEM((1, _K), jnp.int32)] * 4
            + [pltpu.VMEM((_K, _ROW), jnp.float32)] * 3
            + [pltpu.VMEM((_K, _DROW), jnp.float32)] * 3
            + [pltpu.VMEM((_K * 8,), jnp.float32),
               pltpu.VMEM_SHARED((_NP, _ROW), jnp.float32)]
            + [pltpu.SemaphoreType.DMA] * 10
        ),
    )(_edge_kernel_body)
    return fn(xsrc, dpk, src, dst.reshape(_E // _K, _K))


def _combine_body(h_ref, xsrc_ref, a0_ref, a1_ref, b_ref, out_ref):
    xs = xsrc_ref[...]
    a0 = a0_ref[...]
    a1 = a1_ref[...]
    ess = jnp.exp(jnp.clip(xs[:, _F + _H:_F + 2 * _H], -_CLIP, _CLIP))
    essb = jnp.broadcast_to(ess[:, :, None], (_BN, _H, _DH)).reshape(_BN, _F)
    num = essb * xs[:, :_F] + a0[:, :_F] + a1[:, :_F]
    den = ess + a0[:, _F:_F + _H] + a1[:, _F:_F + _H]
    denb = jnp.broadcast_to(den[:, :, None], (_BN, _H, _DH)).reshape(_BN, _F)
    o = num / denb + b_ref[...]
    o = jnp.where(o > 0, o, jnp.exp(jnp.minimum(o, 0.0)) - 1.0)
    out_ref[...] = h_ref[...] + o


def _combine(h, xsrc, acc0, acc1, b):
    nb = _N // _BN
    return pl.pallas_call(
        _combine_body,
        grid=(nb,),
        in_specs=[
            pl.BlockSpec((_BN, _F), lambda i: (i, 0)),
            pl.BlockSpec((_BN, _ROW), lambda i: (i, 0)),
            pl.BlockSpec((_BN, _ROW), lambda i: (i, 0)),
            pl.BlockSpec((_BN, _ROW), lambda i: (i, 0)),
            pl.BlockSpec((1, _F), lambda i: (0, 0)),
        ],
        out_specs=pl.BlockSpec((_BN, _F), lambda i: (i, 0)),
        out_shape=jax.ShapeDtypeStruct((_N, _F), jnp.float32),
    )(h, xsrc, acc0, acc1, b)


def kernel(x, persona, edge_index, gate_W, gate_b, persona_W, lin_W, att_W, bias):
    src = edge_index[0]
    dst = edge_index[1]
    h = x
    L = gate_W.shape[0]
    for l in range(L):
        gw = gate_W[l, :, :, 0].T
        gb = gate_b[l, :, 0][None, :]
        pw = persona_W[l].transpose(1, 0, 2).reshape(_F, _F)
        lw = lin_W[l].transpose(1, 0, 2).reshape(_F, _F)
        aa = att_W[l, :, :_DH, 0].reshape(1, _F)
        ab = att_W[l, :, _DH:, 0].reshape(1, _F)
        bl = bias[l][None, :]
        xsrc, dpk = _project(h, persona, gw, gb, pw, lw, aa, ab)
        acc = _edge_pass(xsrc, dpk, src, dst)
        h = _combine(h, xsrc, acc[0, :_N], acc[1, :_N], bl)
    return h


# speedup vs baseline: 1.6766x; 1.0083x over previous
import functools

import jax
import jax.numpy as jnp
from jax import lax
from jax.experimental import pallas as pl
from jax.experimental.pallas import tpu as pltpu
from jax.experimental.pallas import tpu_sc as plsc

_N = 10000
_H = 4
_DH = 32
_F = _H * _DH
_ROW = 144
_DROW = 16
_NEG = 0.2
_BN = 1000
_NC = 2
_NS = 16
_K = 80
_NP = 10000
_RPT = _NP // _NS
_E = 320000
_CLIP = 60.0


def _lrelu(v):
    return jnp.where(v >= 0, v, _NEG * v)


def _project_body(h_ref, p_ref, gw_ref, gb_ref, pw_ref, lw_ref, aa_ref, ab_ref,
                  xsrc_ref, dpk_ref):
    hb = h_ref[...]
    pb = p_ref[...]
    g = jnp.dot(hb, gw_ref[...], preferred_element_type=jnp.float32) + gb_ref[...]
    pf = jnp.dot(pb, pw_ref[...], preferred_element_type=jnp.float32)
    xf = jnp.dot(hb, lw_ref[...], preferred_element_type=jnp.float32)
    ai = jnp.sum((pf * aa_ref[...]).reshape(_BN, _H, _DH), axis=-1) * g
    aj = jnp.sum((pf * ab_ref[...]).reshape(_BN, _H, _DH), axis=-1) * g
    ss = _lrelu(ai + aj)
    z = jnp.zeros((_BN, _ROW - _F - 2 * _H), jnp.float32)
    xsrc_ref[...] = jnp.concatenate([xf, ai, ss, z], axis=1)
    dpk_ref[...] = jnp.concatenate(
        [aj, jnp.zeros((_BN, _DROW - _H), jnp.float32)], axis=1)


def _project(h, persona, gw, gb, pw, lw, aa, ab):
    nb = _N // _BN
    return pl.pallas_call(
        _project_body,
        grid=(nb,),
        in_specs=[
            pl.BlockSpec((_BN, _F), lambda i: (i, 0)),
            pl.BlockSpec((_BN, _F), lambda i: (i, 0)),
            pl.BlockSpec((_F, _H), lambda i: (0, 0)),
            pl.BlockSpec((1, _H), lambda i: (0, 0)),
            pl.BlockSpec((_F, _F), lambda i: (0, 0)),
            pl.BlockSpec((_F, _F), lambda i: (0, 0)),
            pl.BlockSpec((1, _F), lambda i: (0, 0)),
            pl.BlockSpec((1, _F), lambda i: (0, 0)),
        ],
        out_specs=[
            pl.BlockSpec((_BN, _ROW), lambda i: (i, 0)),
            pl.BlockSpec((_BN, _DROW), lambda i: (i, 0)),
        ],
        out_shape=[
            jax.ShapeDtypeStruct((_N, _ROW), jnp.float32),
            jax.ShapeDtypeStruct((_N, _DROW), jnp.float32),
        ],
    )(h, persona, gw, gb, pw, lw, aa, ab)


def _edge_kernel_body(xsrc_hbm, dpk_hbm, src_hbm, dst2_hbm, out_hbm,
                      si0, si1, si2, si3, di0, di1, di2, di3,
                      rows0, rows1, rows2, dp0, dp1, dp2,
                      wbuf, acc_sh,
                      is0, is1, is2, is3, gs0, gs1, gs2, ss0, ss1, ss2):
    sidx4 = (si0, si1, si2, si3)
    didx4 = (di0, di1, di2, di3)
    rows3 = (rows0, rows1, rows2)
    dpks3 = (dp0, dp1, dp2)
    isem = (is0, is1, is2, is3)
    gsem = (gs0, gs1, gs2)
    ssem = (ss0, ss1, ss2)
    nch = dst2_hbm.shape[0] // (_NC * _NS)
    ept = nch * _K
    cid = lax.axis_index("c")
    sid = lax.axis_index("s")
    wid = cid * _NS + sid
    lane = jnp.arange(16, dtype=jnp.int32)
    zero16 = jnp.zeros((16,), jnp.float32)

    for j in range(_K * 8 // 16):
        wbuf[pl.ds(j * 16, 16)] = zero16

    def _zb_row(i, _):
        for j in range(_ROW // 16):
            rows0[i, pl.ds(j * 16, 16)] = zero16
        return 0
    lax.fori_loop(0, _K, _zb_row, 0)
    nfull = _RPT // _K
    for r in range(nfull):
        pltpu.sync_copy(
            rows0, acc_sh.at[pl.ds(pl.multiple_of(sid * _RPT + r * _K, 1), _K)])
    rem = _RPT - nfull * _K
    if rem:
        pltpu.sync_copy(
            rows0.at[pl.ds(0, rem)],
            acc_sh.at[pl.ds(pl.multiple_of(sid * _RPT + nfull * _K, 1), rem)])
    plsc.subcore_barrier()

    pat8 = jnp.where(lane < _H, lane, 4).astype(jnp.int32)
    hvec = [jnp.full((16,), h, jnp.int32) for h in range(_H)]
    zvec = jnp.zeros((16,), jnp.int32)
    base_e = wid * ept

    def istart(c, k):
        off = pl.multiple_of(base_e + c * _K, 8)
        pltpu.async_copy(src_hbm.at[pl.ds(off, _K)], sidx4[k], isem[k])
        crow = pl.multiple_of(wid * nch + c, 1)
        pltpu.async_copy(dst2_hbm.at[pl.ds(crow, 1)], didx4[k], isem[k])

    def iwait(c, k):
        off = pl.multiple_of(base_e + c * _K, 8)
        pltpu.make_async_copy(src_hbm.at[pl.ds(off, _K)], sidx4[k], isem[k]).wait()
        crow = pl.multiple_of(wid * nch + c, 1)
        pltpu.make_async_copy(dst2_hbm.at[pl.ds(crow, 1)], didx4[k], isem[k]).wait()

    def gather_start(b, k):
        pltpu.async_copy(xsrc_hbm.at[sidx4[k]], rows3[b], gsem[b])
        pltpu.async_copy(dpk_hbm.at[didx4[k].at[0]], dpks3[b], gsem[b])

    def gather_wait(b, k):
        pltpu.make_async_copy(xsrc_hbm.at[sidx4[k]], rows3[b], gsem[b]).wait()
        pltpu.make_async_copy(dpk_hbm.at[didx4[k].at[0]], dpks3[b], gsem[b]).wait()

    def scatter_start(b, k):
        pltpu.async_copy(rows3[b], acc_sh.at[didx4[k].at[0]], ssem[b], add=True)

    def scatter_wait(b, k):
        pltpu.make_async_copy(rows3[b], acc_sh.at[didx4[k].at[0]], ssem[b]).wait()

    def compute(b, k):
        rows = rows3[b]
        dpks = dpks3[b]
        sidx = sidx4[k]
        didx = didx4[k]

        @plsc.parallel_loop(0, _K // 16)
        def _score(g):
            e16 = g * 16 + lane
            sv = plsc.load_gather(sidx, [e16])
            dv = plsc.load_gather(didx, [zvec, e16])
            m = sv != dv
            for h in range(_H):
                ai = plsc.load_gather(rows, [e16, hvec[h] + _F])
                aj = plsc.load_gather(dpks, [e16, hvec[h]])
                sc = _lrelu(ai + aj)
                w = jnp.exp(jnp.clip(sc, -_CLIP, _CLIP))
                w = jnp.where(m, w, 0.0)
                plsc.store_scatter(wbuf, [e16 * 8 + h], w)

        @plsc.parallel_loop(0, _K, unroll=2)
        def _mul(e):
            for h in range(_H):
                wp = plsc.load_gather(wbuf, [e * 8 + hvec[h]])
                for j in (2 * h, 2 * h + 1):
                    rows[e, pl.ds(j * 16, 16)] = wp * rows[e, pl.ds(j * 16, 16)]
            rows[e, pl.ds(8 * 16, 16)] = plsc.load_gather(wbuf, [e * 8 + pat8])

    def step(c, j, guard0, tail):
        b3, b4 = j % 3, j % 4
        pv3, pv4 = (j + 2) % 3, (j + 3) % 4
        bn3, bn4 = (j + 2) % 3, (j + 2) % 4
        nx4 = (j + 3) % 4
        gather_wait(b3, b4)
        compute(b3, b4)
        scatter_start(b3, b4)
        if guard0:
            @pl.when(c > 0)
            def _():
                scatter_wait(pv3, pv4)
        else:
            scatter_wait(pv3, pv4)
        if not tail or c + 2 < nch:
            iwait(c + 2, bn4)
            gather_start(bn3, bn4)
        if not tail or c + 3 < nch:
            istart(c + 3, nx4)

    nmain = (nch - 5) // 12
    istart(0, 0)
    istart(1, 1)
    istart(2, 2)
    iwait(0, 0)
    gather_start(0, 0)
    iwait(1, 1)
    gather_start(1, 1)

    def _pipe(t, _):
        c0 = t * 12
        for j in range(12):
            step(c0 + j, j, j == 0, False)
        return 0
    lax.fori_loop(0, nmain, _pipe, 0)
    for c in range(nmain * 12, nch):
        step(c, c % 12, False, True)
    scatter_wait((nch - 1) % 3, (nch - 1) % 4)

    plsc.subcore_barrier()
    obase = pl.multiple_of(sid * _RPT, 1)
    pltpu.sync_copy(acc_sh.at[pl.ds(obase, _RPT)],
                    out_hbm.at[cid, pl.ds(obase, _RPT)])


def _edge_pass(xsrc, dpk, src, dst):
    mesh = plsc.VectorSubcoreMesh(core_axis_name="c", subcore_axis_name="s",
                                  num_cores=_NC, num_subcores=_NS)
    fn = functools.partial(
        pl.kernel,
        out_type=jax.ShapeDtypeStruct((_NC, _NP, _ROW), jnp.float32),
        mesh=mesh,
        compiler_params=pltpu.CompilerParams(use_tc_tiling_on_sc=False,
                                             needs_layout_passes=False),
        scratch_types=(
            [pltpu.VMEM((_K,), jnp.int32)] * 4
            + [pltpu.VMEM((1, _K), jnp.int32)] * 4
            + [pltpu.VMEM((_K, _ROW), jnp.float32)] * 3
            + [pltpu.VMEM((_K, _DROW), jnp.float32)] * 3
            + [pltpu.VMEM((_K * 8,), jnp.float32),
               pltpu.VMEM_SHARED((_NP, _ROW), jnp.float32)]
            + [pltpu.SemaphoreType.DMA] * 10
        ),
    )(_edge_kernel_body)
    return fn(xsrc, dpk, src, dst.reshape(_E // _K, _K))


def _comb_proj_body(h_ref, xsp_ref, a0_ref, a1_ref, b_ref, p_ref,
                    gw_ref, gb_ref, pw_ref, lw_ref, aa_ref, ab_ref,
                    out_ref, xsrc_ref, dpk_ref):
    xs = xsp_ref[...]
    a0 = a0_ref[...]
    a1 = a1_ref[...]
    ess = jnp.exp(jnp.clip(xs[:, _F + _H:_F + 2 * _H], -_CLIP, _CLIP))
    essb = jnp.broadcast_to(ess[:, :, None], (_BN, _H, _DH)).reshape(_BN, _F)
    num = essb * xs[:, :_F] + a0[:, :_F] + a1[:, :_F]
    den = ess + a0[:, _F:_F + _H] + a1[:, _F:_F + _H]
    denb = jnp.broadcast_to(den[:, :, None], (_BN, _H, _DH)).reshape(_BN, _F)
    o = num / denb + b_ref[...]
    o = jnp.where(o > 0, o, jnp.exp(jnp.minimum(o, 0.0)) - 1.0)
    hb = h_ref[...] + o
    out_ref[...] = hb
    pb = p_ref[...]
    g = jnp.dot(hb, gw_ref[...], preferred_element_type=jnp.float32) + gb_ref[...]
    pf = jnp.dot(pb, pw_ref[...], preferred_element_type=jnp.float32)
    xf = jnp.dot(hb, lw_ref[...], preferred_element_type=jnp.float32)
    ai = jnp.sum((pf * aa_ref[...]).reshape(_BN, _H, _DH), axis=-1) * g
    aj = jnp.sum((pf * ab_ref[...]).reshape(_BN, _H, _DH), axis=-1) * g
    ss = _lrelu(ai + aj)
    z = jnp.zeros((_BN, _ROW - _F - 2 * _H), jnp.float32)
    xsrc_ref[...] = jnp.concatenate([xf, ai, ss, z], axis=1)
    dpk_ref[...] = jnp.concatenate(
        [aj, jnp.zeros((_BN, _DROW - _H), jnp.float32)], axis=1)


def _comb_proj(h, xsp, acc0, acc1, b, persona, gw, gb, pw, lw, aa, ab):
    nb = _N // _BN
    blk = lambda r, c: pl.BlockSpec((r, c), lambda i: (i, 0))
    wspec = lambda r, c: pl.BlockSpec((r, c), lambda i: (0, 0))
    return pl.pallas_call(
        _comb_proj_body,
        grid=(nb,),
        in_specs=[
            blk(_BN, _F), blk(_BN, _ROW), blk(_BN, _ROW), blk(_BN, _ROW),
            wspec(1, _F), blk(_BN, _F),
            wspec(_F, _H), wspec(1, _H), wspec(_F, _F), wspec(_F, _F),
            wspec(1, _F), wspec(1, _F),
        ],
        out_specs=[blk(_BN, _F), blk(_BN, _ROW), blk(_BN, _DROW)],
        out_shape=[
            jax.ShapeDtypeStruct((_N, _F), jnp.float32),
            jax.ShapeDtypeStruct((_N, _ROW), jnp.float32),
            jax.ShapeDtypeStruct((_N, _DROW), jnp.float32),
        ],
    )(h, xsp, acc0, acc1, b, persona, gw, gb, pw, lw, aa, ab)


def _combine_body(h_ref, xsrc_ref, a0_ref, a1_ref, b_ref, out_ref):
    xs = xsrc_ref[...]
    a0 = a0_ref[...]
    a1 = a1_ref[...]
    ess = jnp.exp(jnp.clip(xs[:, _F + _H:_F + 2 * _H], -_CLIP, _CLIP))
    essb = jnp.broadcast_to(ess[:, :, None], (_BN, _H, _DH)).reshape(_BN, _F)
    num = essb * xs[:, :_F] + a0[:, :_F] + a1[:, :_F]
    den = ess + a0[:, _F:_F + _H] + a1[:, _F:_F + _H]
    denb = jnp.broadcast_to(den[:, :, None], (_BN, _H, _DH)).reshape(_BN, _F)
    o = num / denb + b_ref[...]
    o = jnp.where(o > 0, o, jnp.exp(jnp.minimum(o, 0.0)) - 1.0)
    out_ref[...] = h_ref[...] + o


def _combine(h, xsrc, acc0, acc1, b):
    nb = _N // _BN
    return pl.pallas_call(
        _combine_body,
        grid=(nb,),
        in_specs=[
            pl.BlockSpec((_BN, _F), lambda i: (i, 0)),
            pl.BlockSpec((_BN, _ROW), lambda i: (i, 0)),
            pl.BlockSpec((_BN, _ROW), lambda i: (i, 0)),
            pl.BlockSpec((_BN, _ROW), lambda i: (i, 0)),
            pl.BlockSpec((1, _F), lambda i: (0, 0)),
        ],
        out_specs=pl.BlockSpec((_BN, _F), lambda i: (i, 0)),
        out_shape=jax.ShapeDtypeStruct((_N, _F), jnp.float32),
    )(h, xsrc, acc0, acc1, b)


def kernel(x, persona, edge_index, gate_W, gate_b, persona_W, lin_W, att_W, bias):
    src = edge_index[0]
    dst = edge_index[1]

    def wts(l):
        gw = gate_W[l, :, :, 0].T
        gb = gate_b[l, :, 0][None, :]
        pw = persona_W[l].transpose(1, 0, 2).reshape(_F, _F)
        lw = lin_W[l].transpose(1, 0, 2).reshape(_F, _F)
        aa = att_W[l, :, :_DH, 0].reshape(1, _F)
        ab = att_W[l, :, _DH:, 0].reshape(1, _F)
        return gw, gb, pw, lw, aa, ab

    xsrc, dpk = _project(x, persona, *wts(0))
    acc = _edge_pass(xsrc, dpk, src, dst)
    h, xsrc, dpk = _comb_proj(x, xsrc, acc[0, :_N], acc[1, :_N],
                              bias[0][None, :], persona, *wts(1))
    acc = _edge_pass(xsrc, dpk, src, dst)
    return _combine(h, xsrc, acc[0, :_N], acc[1, :_N], bias[1][None, :])


# speedup vs baseline: 1.7508x; 1.0443x over previous
import functools

import jax
import jax.numpy as jnp
from jax import lax
from jax.experimental import pallas as pl
from jax.experimental.pallas import tpu as pltpu
from jax.experimental.pallas import tpu_sc as plsc

_N = 10000
_H = 4
_DH = 32
_F = _H * _DH
_ROW = 144
_DROW = 16
_NEG = 0.2
_BN = 2000
_NC = 2
_NS = 16
_K = 80
_NP = 10000
_RPT = _NP // _NS
_E = 320000
_CLIP = 60.0


def _lrelu(v):
    return jnp.where(v >= 0, v, _NEG * v)


def _project_body(h_ref, p_ref, gw_ref, gb_ref, pw_ref, lw_ref, aa_ref, ab_ref,
                  xsrc_ref, dpk_ref):
    hb = h_ref[...]
    pb = p_ref[...]
    g = jnp.dot(hb, gw_ref[...], preferred_element_type=jnp.float32) + gb_ref[...]
    pf = jnp.dot(pb, pw_ref[...], preferred_element_type=jnp.float32)
    xf = jnp.dot(hb, lw_ref[...], preferred_element_type=jnp.float32)
    ai = jnp.sum((pf * aa_ref[...]).reshape(_BN, _H, _DH), axis=-1) * g
    aj = jnp.sum((pf * ab_ref[...]).reshape(_BN, _H, _DH), axis=-1) * g
    ss = _lrelu(ai + aj)
    z = jnp.zeros((_BN, _ROW - _F - 2 * _H), jnp.float32)
    xsrc_ref[...] = jnp.concatenate([xf, ai, ss, z], axis=1)
    dpk_ref[...] = jnp.concatenate(
        [aj, jnp.zeros((_BN, _DROW - _H), jnp.float32)], axis=1)


def _project(h, persona, gw, gb, pw, lw, aa, ab):
    nb = _N // _BN
    return pl.pallas_call(
        _project_body,
        grid=(nb,),
        in_specs=[
            pl.BlockSpec((_BN, _F), lambda i: (i, 0)),
            pl.BlockSpec((_BN, _F), lambda i: (i, 0)),
            pl.BlockSpec((_F, _H), lambda i: (0, 0)),
            pl.BlockSpec((1, _H), lambda i: (0, 0)),
            pl.BlockSpec((_F, _F), lambda i: (0, 0)),
            pl.BlockSpec((_F, _F), lambda i: (0, 0)),
            pl.BlockSpec((1, _F), lambda i: (0, 0)),
            pl.BlockSpec((1, _F), lambda i: (0, 0)),
        ],
        out_specs=[
            pl.BlockSpec((_BN, _ROW), lambda i: (i, 0)),
            pl.BlockSpec((_BN, _DROW), lambda i: (i, 0)),
        ],
        out_shape=[
            jax.ShapeDtypeStruct((_N, _ROW), jnp.float32),
            jax.ShapeDtypeStruct((_N, _DROW), jnp.float32),
        ],
    )(h, persona, gw, gb, pw, lw, aa, ab)


def _edge_kernel_body(xsrc_hbm, dpk_hbm, src_hbm, dst2_hbm, out_hbm,
                      si0, si1, si2, si3, di0, di1, di2, di3,
                      rows0, rows1, rows2, dp0, dp1, dp2,
                      wbuf, acc_sh,
                      is0, is1, is2, is3, gs0, gs1, gs2, ss0, ss1, ss2):
    sidx4 = (si0, si1, si2, si3)
    didx4 = (di0, di1, di2, di3)
    rows3 = (rows0, rows1, rows2)
    dpks3 = (dp0, dp1, dp2)
    isem = (is0, is1, is2, is3)
    gsem = (gs0, gs1, gs2)
    ssem = (ss0, ss1, ss2)
    nch = dst2_hbm.shape[0] // (_NC * _NS)
    ept = nch * _K
    cid = lax.axis_index("c")
    sid = lax.axis_index("s")
    wid = cid * _NS + sid
    lane = jnp.arange(16, dtype=jnp.int32)
    zero16 = jnp.zeros((16,), jnp.float32)

    for j in range(_K * 8 // 16):
        wbuf[pl.ds(j * 16, 16)] = zero16

    def _zb_row(i, _):
        for j in range(_ROW // 16):
            rows0[i, pl.ds(j * 16, 16)] = zero16
        return 0
    lax.fori_loop(0, _K, _zb_row, 0)
    nfull = _RPT // _K
    for r in range(nfull):
        pltpu.sync_copy(
            rows0, acc_sh.at[pl.ds(pl.multiple_of(sid * _RPT + r * _K, 1), _K)])
    rem = _RPT - nfull * _K
    if rem:
        pltpu.sync_copy(
            rows0.at[pl.ds(0, rem)],
            acc_sh.at[pl.ds(pl.multiple_of(sid * _RPT + nfull * _K, 1), rem)])
    plsc.subcore_barrier()

    pat8 = jnp.where(lane < _H, lane, 4).astype(jnp.int32)
    hvec = [jnp.full((16,), h, jnp.int32) for h in range(_H)]
    zvec = jnp.zeros((16,), jnp.int32)
    base_e = wid * ept

    def istart(c, k):
        off = pl.multiple_of(base_e + c * _K, 8)
        pltpu.async_copy(src_hbm.at[pl.ds(off, _K)], sidx4[k], isem[k])
        crow = pl.multiple_of(wid * nch + c, 1)
        pltpu.async_copy(dst2_hbm.at[pl.ds(crow, 1)], didx4[k], isem[k])

    def iwait(c, k):
        off = pl.multiple_of(base_e + c * _K, 8)
        pltpu.make_async_copy(src_hbm.at[pl.ds(off, _K)], sidx4[k], isem[k]).wait()
        crow = pl.multiple_of(wid * nch + c, 1)
        pltpu.make_async_copy(dst2_hbm.at[pl.ds(crow, 1)], didx4[k], isem[k]).wait()

    def gather_start(b, k):
        pltpu.async_copy(xsrc_hbm.at[sidx4[k]], rows3[b], gsem[b])
        pltpu.async_copy(dpk_hbm.at[didx4[k].at[0]], dpks3[b], gsem[b])

    def gather_wait(b, k):
        pltpu.make_async_copy(xsrc_hbm.at[sidx4[k]], rows3[b], gsem[b]).wait()
        pltpu.make_async_copy(dpk_hbm.at[didx4[k].at[0]], dpks3[b], gsem[b]).wait()

    def scatter_start(b, k):
        pltpu.async_copy(rows3[b], acc_sh.at[didx4[k].at[0]], ssem[b], add=True)

    def scatter_wait(b, k):
        pltpu.make_async_copy(rows3[b], acc_sh.at[didx4[k].at[0]], ssem[b]).wait()

    def compute(b, k):
        rows = rows3[b]
        dpks = dpks3[b]
        sidx = sidx4[k]
        didx = didx4[k]

        @plsc.parallel_loop(0, _K // 16)
        def _score(g):
            e16 = g * 16 + lane
            sv = plsc.load_gather(sidx, [e16])
            dv = plsc.load_gather(didx, [zvec, e16])
            m = sv != dv
            for h in range(_H):
                ai = plsc.load_gather(rows, [e16, hvec[h] + _F])
                aj = plsc.load_gather(dpks, [e16, hvec[h]])
                sc = _lrelu(ai + aj)
                w = jnp.exp(jnp.clip(sc, -_CLIP, _CLIP))
                w = jnp.where(m, w, 0.0)
                plsc.store_scatter(wbuf, [e16 * 8 + h], w)

        @plsc.parallel_loop(0, _K, unroll=2)
        def _mul(e):
            for h in range(_H):
                wp = plsc.load_gather(wbuf, [e * 8 + hvec[h]])
                for j in (2 * h, 2 * h + 1):
                    rows[e, pl.ds(j * 16, 16)] = wp * rows[e, pl.ds(j * 16, 16)]
            rows[e, pl.ds(8 * 16, 16)] = plsc.load_gather(wbuf, [e * 8 + pat8])

    def step(c, j, guard0, tail):
        b3, b4 = j % 3, j % 4
        pv3, pv4 = (j + 2) % 3, (j + 3) % 4
        bn3, bn4 = (j + 2) % 3, (j + 2) % 4
        nx4 = (j + 3) % 4
        gather_wait(b3, b4)
        compute(b3, b4)
        scatter_start(b3, b4)
        if guard0:
            @pl.when(c > 0)
            def _():
                scatter_wait(pv3, pv4)
        else:
            scatter_wait(pv3, pv4)
        if not tail or c + 2 < nch:
            iwait(c + 2, bn4)
            gather_start(bn3, bn4)
        if not tail or c + 3 < nch:
            istart(c + 3, nx4)

    nmain = (nch - 5) // 12
    istart(0, 0)
    istart(1, 1)
    istart(2, 2)
    iwait(0, 0)
    gather_start(0, 0)
    iwait(1, 1)
    gather_start(1, 1)

    def _pipe(t, _):
        c0 = t * 12
        for j in range(12):
            step(c0 + j, j, j == 0, False)
        return 0
    lax.fori_loop(0, nmain, _pipe, 0)
    for c in range(nmain * 12, nch):
        step(c, c % 12, False, True)
    scatter_wait((nch - 1) % 3, (nch - 1) % 4)

    plsc.subcore_barrier()
    obase = pl.multiple_of(sid * _RPT, 1)
    pltpu.sync_copy(acc_sh.at[pl.ds(obase, _RPT)],
                    out_hbm.at[cid, pl.ds(obase, _RPT)])


def _edge_pass(xsrc, dpk, src, dst):
    mesh = plsc.VectorSubcoreMesh(core_axis_name="c", subcore_axis_name="s",
                                  num_cores=_NC, num_subcores=_NS)
    fn = functools.partial(
        pl.kernel,
        out_type=jax.ShapeDtypeStruct((_NC, _NP, _ROW), jnp.float32),
        mesh=mesh,
        compiler_params=pltpu.CompilerParams(use_tc_tiling_on_sc=False,
                                             needs_layout_passes=False),
        scratch_types=(
            [pltpu.VMEM((_K,), jnp.int32)] * 4
            + [pltpu.VMEM((1, _K), jnp.int32)] * 4
            + [pltpu.VMEM((_K, _ROW), jnp.float32)] * 3
            + [pltpu.VMEM((_K, _DROW), jnp.float32)] * 3
            + [pltpu.VMEM((_K * 8,), jnp.float32),
               pltpu.VMEM_SHARED((_NP, _ROW), jnp.float32)]
            + [pltpu.SemaphoreType.DMA] * 10
        ),
    )(_edge_kernel_body)
    return fn(xsrc, dpk, src, dst.reshape(_E // _K, _K))


def _comb_proj_body(h_ref, xsp_ref, a0_ref, a1_ref, b_ref, p_ref,
                    gw_ref, gb_ref, pw_ref, lw_ref, aa_ref, ab_ref,
                    out_ref, xsrc_ref, dpk_ref):
    xs = xsp_ref[...]
    a0 = a0_ref[0]
    a1 = a1_ref[0]
    ess = jnp.exp(jnp.clip(xs[:, _F + _H:_F + 2 * _H], -_CLIP, _CLIP))
    essb = jnp.broadcast_to(ess[:, :, None], (_BN, _H, _DH)).reshape(_BN, _F)
    num = essb * xs[:, :_F] + a0[:, :_F] + a1[:, :_F]
    den = ess + a0[:, _F:_F + _H] + a1[:, _F:_F + _H]
    denb = jnp.broadcast_to(den[:, :, None], (_BN, _H, _DH)).reshape(_BN, _F)
    o = num / denb + b_ref[...]
    o = jnp.where(o > 0, o, jnp.exp(jnp.minimum(o, 0.0)) - 1.0)
    hb = h_ref[...] + o
    out_ref[...] = hb
    pb = p_ref[...]
    g = jnp.dot(hb, gw_ref[...], preferred_element_type=jnp.float32) + gb_ref[...]
    pf = jnp.dot(pb, pw_ref[...], preferred_element_type=jnp.float32)
    xf = jnp.dot(hb, lw_ref[...], preferred_element_type=jnp.float32)
    ai = jnp.sum((pf * aa_ref[...]).reshape(_BN, _H, _DH), axis=-1) * g
    aj = jnp.sum((pf * ab_ref[...]).reshape(_BN, _H, _DH), axis=-1) * g
    ss = _lrelu(ai + aj)
    z = jnp.zeros((_BN, _ROW - _F - 2 * _H), jnp.float32)
    xsrc_ref[...] = jnp.concatenate([xf, ai, ss, z], axis=1)
    dpk_ref[...] = jnp.concatenate(
        [aj, jnp.zeros((_BN, _DROW - _H), jnp.float32)], axis=1)


def _comb_proj(h, xsp, acc, b, persona, gw, gb, pw, lw, aa, ab):
    nb = _N // _BN
    blk = lambda r, c: pl.BlockSpec((r, c), lambda i: (i, 0))
    wspec = lambda r, c: pl.BlockSpec((r, c), lambda i: (0, 0))
    return pl.pallas_call(
        _comb_proj_body,
        grid=(nb,),
        in_specs=[
            blk(_BN, _F), blk(_BN, _ROW),
            pl.BlockSpec((1, _BN, _ROW), lambda i: (0, i, 0)),
            pl.BlockSpec((1, _BN, _ROW), lambda i: (1, i, 0)),
            wspec(1, _F), blk(_BN, _F),
            wspec(_F, _H), wspec(1, _H), wspec(_F, _F), wspec(_F, _F),
            wspec(1, _F), wspec(1, _F),
        ],
        out_specs=[blk(_BN, _F), blk(_BN, _ROW), blk(_BN, _DROW)],
        out_shape=[
            jax.ShapeDtypeStruct((_N, _F), jnp.float32),
            jax.ShapeDtypeStruct((_N, _ROW), jnp.float32),
            jax.ShapeDtypeStruct((_N, _DROW), jnp.float32),
        ],
    )(h, xsp, acc, acc, b, persona, gw, gb, pw, lw, aa, ab)


def _combine_body(h_ref, xsrc_ref, a0_ref, a1_ref, b_ref, out_ref):
    xs = xsrc_ref[...]
    a0 = a0_ref[0]
    a1 = a1_ref[0]
    ess = jnp.exp(jnp.clip(xs[:, _F + _H:_F + 2 * _H], -_CLIP, _CLIP))
    essb = jnp.broadcast_to(ess[:, :, None], (_BN, _H, _DH)).reshape(_BN, _F)
    num = essb * xs[:, :_F] + a0[:, :_F] + a1[:, :_F]
    den = ess + a0[:, _F:_F + _H] + a1[:, _F:_F + _H]
    denb = jnp.broadcast_to(den[:, :, None], (_BN, _H, _DH)).reshape(_BN, _F)
    o = num / denb + b_ref[...]
    o = jnp.where(o > 0, o, jnp.exp(jnp.minimum(o, 0.0)) - 1.0)
    out_ref[...] = h_ref[...] + o


def _combine(h, xsrc, acc, b):
    nb = _N // _BN
    return pl.pallas_call(
        _combine_body,
        grid=(nb,),
        in_specs=[
            pl.BlockSpec((_BN, _F), lambda i: (i, 0)),
            pl.BlockSpec((_BN, _ROW), lambda i: (i, 0)),
            pl.BlockSpec((1, _BN, _ROW), lambda i: (0, i, 0)),
            pl.BlockSpec((1, _BN, _ROW), lambda i: (1, i, 0)),
            pl.BlockSpec((1, _F), lambda i: (0, 0)),
        ],
        out_specs=pl.BlockSpec((_BN, _F), lambda i: (i, 0)),
        out_shape=jax.ShapeDtypeStruct((_N, _F), jnp.float32),
    )(h, xsrc, acc, acc, b)


def kernel(x, persona, edge_index, gate_W, gate_b, persona_W, lin_W, att_W, bias):
    src = edge_index[0]
    dst = edge_index[1]

    def wts(l):
        gw = gate_W[l, :, :, 0].T
        gb = gate_b[l, :, 0][None, :]
        pw = persona_W[l].transpose(1, 0, 2).reshape(_F, _F)
        lw = lin_W[l].transpose(1, 0, 2).reshape(_F, _F)
        aa = att_W[l, :, :_DH, 0].reshape(1, _F)
        ab = att_W[l, :, _DH:, 0].reshape(1, _F)
        return gw, gb, pw, lw, aa, ab

    xsrc, dpk = _project(x, persona, *wts(0))
    acc = _edge_pass(xsrc, dpk, src, dst)
    h, xsrc, dpk = _comb_proj(x, xsrc, acc, bias[0][None, :], persona, *wts(1))
    acc = _edge_pass(xsrc, dpk, src, dst)
    return _combine(h, xsrc, acc, bias[1][None, :])


# speedup vs baseline: 2.3759x; 1.3570x over previous
import functools

import jax
import jax.numpy as jnp
from jax import lax
from jax.experimental import pallas as pl
from jax.experimental.pallas import tpu as pltpu
from jax.experimental.pallas import tpu_sc as plsc

_N = 10000
_H = 4
_DH = 32
_F = _H * _DH
_ROW = 144
_DROW = 16
_NEG = 0.2
_BN = 2000
_NC = 2
_NS = 16
_K = 80
_NP = 10000
_RPT = _NP // _NS
_E = 320000
_CLIP = 60.0


def _lrelu(v):
    return jnp.where(v >= 0, v, _NEG * v)


def _project_body(h_ref, p_ref, gw_ref, gb_ref, pw_ref, lw_ref, aa_ref, ab_ref,
                  xsrc_ref, dpk_ref):
    hb = h_ref[...]
    pb = p_ref[...]
    g = jnp.dot(hb, gw_ref[...], preferred_element_type=jnp.float32) + gb_ref[...]
    pf = jnp.dot(pb, pw_ref[...], preferred_element_type=jnp.float32)
    xf = jnp.dot(hb, lw_ref[...], preferred_element_type=jnp.float32)
    ai = jnp.dot(pf, aa_ref[...], preferred_element_type=jnp.float32) * g
    aj = jnp.dot(pf, ab_ref[...], preferred_element_type=jnp.float32) * g
    ss = _lrelu(ai + aj)
    z = jnp.zeros((_BN, _ROW - _F - 2 * _H), jnp.float32)
    xsrc_ref[...] = jnp.concatenate([xf, ai, ss, z], axis=1)
    dpk_ref[...] = jnp.concatenate(
        [aj, jnp.zeros((_BN, _DROW - _H), jnp.float32)], axis=1)


def _project(h, persona, gw, gb, pw, lw, aa, ab):
    nb = _N // _BN
    return pl.pallas_call(
        _project_body,
        grid=(nb,),
        in_specs=[
            pl.BlockSpec((_BN, _F), lambda i: (i, 0)),
            pl.BlockSpec((_BN, _F), lambda i: (i, 0)),
            pl.BlockSpec((_F, _H), lambda i: (0, 0)),
            pl.BlockSpec((1, _H), lambda i: (0, 0)),
            pl.BlockSpec((_F, _F), lambda i: (0, 0)),
            pl.BlockSpec((_F, _F), lambda i: (0, 0)),
            pl.BlockSpec((_F, _H), lambda i: (0, 0)),
            pl.BlockSpec((_F, _H), lambda i: (0, 0)),
        ],
        out_specs=[
            pl.BlockSpec((_BN, _ROW), lambda i: (i, 0)),
            pl.BlockSpec((_BN, _DROW), lambda i: (i, 0)),
        ],
        out_shape=[
            jax.ShapeDtypeStruct((_N, _ROW), jnp.float32),
            jax.ShapeDtypeStruct((_N, _DROW), jnp.float32),
        ],
    )(h, persona, gw, gb, pw, lw, aa, ab)


def _edge_kernel_body(xsrc_hbm, dpk_hbm, src_hbm, dst2_hbm, out_hbm,
                      si0, si1, si2, si3, di0, di1, di2, di3,
                      rows0, rows1, rows2, dp0, dp1, dp2,
                      wbuf, acc_sh,
                      is0, is1, is2, is3, gs0, gs1, gs2, ss0, ss1, ss2):
    sidx4 = (si0, si1, si2, si3)
    didx4 = (di0, di1, di2, di3)
    rows3 = (rows0, rows1, rows2)
    dpks3 = (dp0, dp1, dp2)
    isem = (is0, is1, is2, is3)
    gsem = (gs0, gs1, gs2)
    ssem = (ss0, ss1, ss2)
    nch = dst2_hbm.shape[0] // (_NC * _NS)
    ept = nch * _K
    cid = lax.axis_index("c")
    sid = lax.axis_index("s")
    wid = cid * _NS + sid
    lane = jnp.arange(16, dtype=jnp.int32)
    zero16 = jnp.zeros((16,), jnp.float32)

    for j in range(_K * 8 // 16):
        wbuf[pl.ds(j * 16, 16)] = zero16

    def _zb_row(i, _):
        for j in range(_ROW // 16):
            rows0[i, pl.ds(j * 16, 16)] = zero16
        return 0
    lax.fori_loop(0, _K, _zb_row, 0)
    nfull = _RPT // _K
    for r in range(nfull):
        pltpu.sync_copy(
            rows0, acc_sh.at[pl.ds(pl.multiple_of(sid * _RPT + r * _K, 1), _K)])
    rem = _RPT - nfull * _K
    if rem:
        pltpu.sync_copy(
            rows0.at[pl.ds(0, rem)],
            acc_sh.at[pl.ds(pl.multiple_of(sid * _RPT + nfull * _K, 1), rem)])
    plsc.subcore_barrier()

    pat8 = jnp.where(lane < _H, lane, 4).astype(jnp.int32)
    hvec = [jnp.full((16,), h, jnp.int32) for h in range(_H)]
    zvec = jnp.zeros((16,), jnp.int32)
    base_e = wid * ept

    def istart(c, k):
        off = pl.multiple_of(base_e + c * _K, 8)
        pltpu.async_copy(src_hbm.at[pl.ds(off, _K)], sidx4[k], isem[k])
        crow = pl.multiple_of(wid * nch + c, 1)
        pltpu.async_copy(dst2_hbm.at[pl.ds(crow, 1)], didx4[k], isem[k])

    def iwait(c, k):
        off = pl.multiple_of(base_e + c * _K, 8)
        pltpu.make_async_copy(src_hbm.at[pl.ds(off, _K)], sidx4[k], isem[k]).wait()
        crow = pl.multiple_of(wid * nch + c, 1)
        pltpu.make_async_copy(dst2_hbm.at[pl.ds(crow, 1)], didx4[k], isem[k]).wait()

    def gather_start(b, k):
        pltpu.async_copy(xsrc_hbm.at[sidx4[k]], rows3[b], gsem[b])
        pltpu.async_copy(dpk_hbm.at[didx4[k].at[0]], dpks3[b], gsem[b])

    def gather_wait(b, k):
        pltpu.make_async_copy(xsrc_hbm.at[sidx4[k]], rows3[b], gsem[b]).wait()
        pltpu.make_async_copy(dpk_hbm.at[didx4[k].at[0]], dpks3[b], gsem[b]).wait()

    def scatter_start(b, k):
        pltpu.async_copy(rows3[b], acc_sh.at[didx4[k].at[0]], ssem[b], add=True)

    def scatter_wait(b, k):
        pltpu.make_async_copy(rows3[b], acc_sh.at[didx4[k].at[0]], ssem[b]).wait()

    def compute(b, k):
        rows = rows3[b]
        dpks = dpks3[b]
        sidx = sidx4[k]
        didx = didx4[k]

        @plsc.parallel_loop(0, _K // 16)
        def _score(g):
            e16 = g * 16 + lane
            sv = plsc.load_gather(sidx, [e16])
            dv = plsc.load_gather(didx, [zvec, e16])
            m = sv != dv
            for h in range(_H):
                ai = plsc.load_gather(rows, [e16, hvec[h] + _F])
                aj = plsc.load_gather(dpks, [e16, hvec[h]])
                sc = _lrelu(ai + aj)
                w = jnp.exp(jnp.clip(sc, -_CLIP, _CLIP))
                w = jnp.where(m, w, 0.0)
                plsc.store_scatter(wbuf, [e16 * 8 + h], w)

        @plsc.parallel_loop(0, _K, unroll=2)
        def _mul(e):
            for h in range(_H):
                wp = plsc.load_gather(wbuf, [e * 8 + hvec[h]])
                for j in (2 * h, 2 * h + 1):
                    rows[e, pl.ds(j * 16, 16)] = wp * rows[e, pl.ds(j * 16, 16)]
            rows[e, pl.ds(8 * 16, 16)] = plsc.load_gather(wbuf, [e * 8 + pat8])

    def step(c, j, guard0, tail):
        b3, b4 = j % 3, j % 4
        pv3, pv4 = (j + 2) % 3, (j + 3) % 4
        bn3, bn4 = (j + 2) % 3, (j + 2) % 4
        nx4 = (j + 3) % 4
        gather_wait(b3, b4)
        compute(b3, b4)
        scatter_start(b3, b4)
        if guard0:
            @pl.when(c > 0)
            def _():
                scatter_wait(pv3, pv4)
        else:
            scatter_wait(pv3, pv4)
        if not tail or c + 2 < nch:
            iwait(c + 2, bn4)
            gather_start(bn3, bn4)
        if not tail or c + 3 < nch:
            istart(c + 3, nx4)

    nmain = (nch - 5) // 12
    istart(0, 0)
    istart(1, 1)
    istart(2, 2)
    iwait(0, 0)
    gather_start(0, 0)
    iwait(1, 1)
    gather_start(1, 1)

    def _pipe(t, _):
        c0 = t * 12
        for j in range(12):
            step(c0 + j, j, j == 0, False)
        return 0
    lax.fori_loop(0, nmain, _pipe, 0)
    for c in range(nmain * 12, nch):
        step(c, c % 12, False, True)
    scatter_wait((nch - 1) % 3, (nch - 1) % 4)

    plsc.subcore_barrier()
    obase = pl.multiple_of(sid * _RPT, 1)
    pltpu.sync_copy(acc_sh.at[pl.ds(obase, _RPT)],
                    out_hbm.at[cid, pl.ds(obase, _RPT)])


def _edge_pass(xsrc, dpk, src, dst):
    mesh = plsc.VectorSubcoreMesh(core_axis_name="c", subcore_axis_name="s",
                                  num_cores=_NC, num_subcores=_NS)
    fn = functools.partial(
        pl.kernel,
        out_type=jax.ShapeDtypeStruct((_NC, _NP, _ROW), jnp.float32),
        mesh=mesh,
        compiler_params=pltpu.CompilerParams(use_tc_tiling_on_sc=False,
                                             needs_layout_passes=False),
        scratch_types=(
            [pltpu.VMEM((_K,), jnp.int32)] * 4
            + [pltpu.VMEM((1, _K), jnp.int32)] * 4
            + [pltpu.VMEM((_K, _ROW), jnp.float32)] * 3
            + [pltpu.VMEM((_K, _DROW), jnp.float32)] * 3
            + [pltpu.VMEM((_K * 8,), jnp.float32),
               pltpu.VMEM_SHARED((_NP, _ROW), jnp.float32)]
            + [pltpu.SemaphoreType.DMA] * 10
        ),
    )(_edge_kernel_body)
    return fn(xsrc, dpk, src, dst.reshape(_E // _K, _K))


def _comb_proj_body(h_ref, xsp_ref, a0_ref, a1_ref, b_ref, p_ref,
                    gw_ref, gb_ref, pw_ref, lw_ref, aa_ref, ab_ref, sel_ref,
                    out_ref, xsrc_ref, dpk_ref):
    xs = xsp_ref[...]
    a0 = a0_ref[0]
    a1 = a1_ref[0]
    sel = sel_ref[...]
    ess = jnp.exp(jnp.clip(xs[:, _F + _H:_F + 2 * _H], -_CLIP, _CLIP))
    essb = jnp.dot(ess, sel, preferred_element_type=jnp.float32)
    num = essb * xs[:, :_F] + a0[:, :_F] + a1[:, :_F]
    den = ess + a0[:, _F:_F + _H] + a1[:, _F:_F + _H]
    denb = jnp.dot(den, sel, preferred_element_type=jnp.float32)
    o = num / denb + b_ref[...]
    o = jnp.where(o > 0, o, jnp.exp(jnp.minimum(o, 0.0)) - 1.0)
    hb = h_ref[...] + o
    out_ref[...] = hb
    pb = p_ref[...]
    g = jnp.dot(hb, gw_ref[...], preferred_element_type=jnp.float32) + gb_ref[...]
    pf = jnp.dot(pb, pw_ref[...], preferred_element_type=jnp.float32)
    xf = jnp.dot(hb, lw_ref[...], preferred_element_type=jnp.float32)
    ai = jnp.dot(pf, aa_ref[...], preferred_element_type=jnp.float32) * g
    aj = jnp.dot(pf, ab_ref[...], preferred_element_type=jnp.float32) * g
    ss = _lrelu(ai + aj)
    z = jnp.zeros((_BN, _ROW - _F - 2 * _H), jnp.float32)
    xsrc_ref[...] = jnp.concatenate([xf, ai, ss, z], axis=1)
    dpk_ref[...] = jnp.concatenate(
        [aj, jnp.zeros((_BN, _DROW - _H), jnp.float32)], axis=1)


def _comb_proj(h, xsp, acc, b, persona, gw, gb, pw, lw, aa, ab, sel):
    nb = _N // _BN
    blk = lambda r, c: pl.BlockSpec((r, c), lambda i: (i, 0))
    wspec = lambda r, c: pl.BlockSpec((r, c), lambda i: (0, 0))
    return pl.pallas_call(
        _comb_proj_body,
        grid=(nb,),
        in_specs=[
            blk(_BN, _F), blk(_BN, _ROW),
            pl.BlockSpec((1, _BN, _ROW), lambda i: (0, i, 0)),
            pl.BlockSpec((1, _BN, _ROW), lambda i: (1, i, 0)),
            wspec(1, _F), blk(_BN, _F),
            wspec(_F, _H), wspec(1, _H), wspec(_F, _F), wspec(_F, _F),
            wspec(_F, _H), wspec(_F, _H), wspec(_H, _F),
        ],
        out_specs=[blk(_BN, _F), blk(_BN, _ROW), blk(_BN, _DROW)],
        out_shape=[
            jax.ShapeDtypeStruct((_N, _F), jnp.float32),
            jax.ShapeDtypeStruct((_N, _ROW), jnp.float32),
            jax.ShapeDtypeStruct((_N, _DROW), jnp.float32),
        ],
    )(h, xsp, acc, acc, b, persona, gw, gb, pw, lw, aa, ab, sel)


def _combine_body(h_ref, xsrc_ref, a0_ref, a1_ref, b_ref, sel_ref, out_ref):
    xs = xsrc_ref[...]
    a0 = a0_ref[0]
    a1 = a1_ref[0]
    sel = sel_ref[...]
    ess = jnp.exp(jnp.clip(xs[:, _F + _H:_F + 2 * _H], -_CLIP, _CLIP))
    essb = jnp.dot(ess, sel, preferred_element_type=jnp.float32)
    num = essb * xs[:, :_F] + a0[:, :_F] + a1[:, :_F]
    den = ess + a0[:, _F:_F + _H] + a1[:, _F:_F + _H]
    denb = jnp.dot(den, sel, preferred_element_type=jnp.float32)
    o = num / denb + b_ref[...]
    o = jnp.where(o > 0, o, jnp.exp(jnp.minimum(o, 0.0)) - 1.0)
    out_ref[...] = h_ref[...] + o


def _combine(h, xsrc, acc, b, sel):
    nb = _N // _BN
    return pl.pallas_call(
        _combine_body,
        grid=(nb,),
        in_specs=[
            pl.BlockSpec((_BN, _F), lambda i: (i, 0)),
            pl.BlockSpec((_BN, _ROW), lambda i: (i, 0)),
            pl.BlockSpec((1, _BN, _ROW), lambda i: (0, i, 0)),
            pl.BlockSpec((1, _BN, _ROW), lambda i: (1, i, 0)),
            pl.BlockSpec((1, _F), lambda i: (0, 0)),
            pl.BlockSpec((_H, _F), lambda i: (0, 0)),
        ],
        out_specs=pl.BlockSpec((_BN, _F), lambda i: (i, 0)),
        out_shape=jax.ShapeDtypeStruct((_N, _F), jnp.float32),
    )(h, xsrc, acc, acc, b, sel)


def kernel(x, persona, edge_index, gate_W, gate_b, persona_W, lin_W, att_W, bias):
    src = edge_index[0]
    dst = edge_index[1]

    eyeh = jnp.repeat(jnp.eye(_H, dtype=jnp.float32), _DH, axis=1)

    def wts(l):
        gw = gate_W[l, :, :, 0].T
        gb = gate_b[l, :, 0][None, :]
        pw = persona_W[l].transpose(1, 0, 2).reshape(_F, _F)
        lw = lin_W[l].transpose(1, 0, 2).reshape(_F, _F)
        aa = eyeh.T * att_W[l, :, :_DH, 0].reshape(_F)[:, None]
        ab = eyeh.T * att_W[l, :, _DH:, 0].reshape(_F)[:, None]
        return gw, gb, pw, lw, aa, ab

    xsrc, dpk = _project(x, persona, *wts(0))
    acc = _edge_pass(xsrc, dpk, src, dst)
    h, xsrc, dpk = _comb_proj(x, xsrc, acc, bias[0][None, :], persona,
                              *wts(1), eyeh)
    acc = _edge_pass(xsrc, dpk, src, dst)
    return _combine(h, xsrc, acc, bias[1][None, :], eyeh)
